# baseline (speedup 1.0000x reference)
import functools

import jax
import jax.numpy as jnp
from jax import lax
from jax.experimental import pallas as pl
from jax.experimental.pallas import tpu as pltpu
from jax.experimental.pallas import tpu_sc as plsc

B, S, D = 16, 2048, 1024
L = 16
SUB = 8
W = D // SUB


def _mid_body(table_hbm, lens_hbm, out_hbm, lens_v, idx_v, rows_v, sem):
    t = lax.axis_index("s")

    pltpu.sync_copy(lens_hbm, lens_v)
    lane = lax.iota(jnp.int32, L)
    my_len = lax.gather(
        lens_v[...],
        jnp.full((L, 1), t, jnp.int32),
        dimension_numbers=lax.GatherDimensionNumbers(
            offset_dims=(), collapsed_slice_dims=(0,), start_index_map=(0,)
        ),
        slice_sizes=(1,),
        mode=lax.GatherScatterMode.PROMISE_IN_BOUNDS,
    )
    idx_v[...] = (t * S + lax.shift_right_logical(my_len, 1)) * SUB + (lane & 7)
    pltpu.async_copy(table_hbm.at[idx_v], rows_v, sem).wait()

    scale = jnp.where(my_len == 0, jnp.float32(0.0), jnp.float32(1.0))
    for i in range(SUB):
        for j in range(W // L):
            rows_v[i, j * L:(j + 1) * L] = rows_v[i, j * L:(j + 1) * L] * scale

    pltpu.sync_copy(rows_v.at[pl.ds(0, SUB)], out_hbm.at[pl.ds(t * SUB, SUB)])


def kernel(payload, seq_lens):
    table = payload.reshape(B * S * SUB, W)
    lens = seq_lens.astype(jnp.int32)
    mesh = plsc.VectorSubcoreMesh(
        core_axis_name="c", subcore_axis_name="s", num_cores=1
    )
    k = functools.partial(
        pl.kernel,
        mesh=mesh,
        out_type=jax.ShapeDtypeStruct((B * SUB, W), jnp.float32),
        scratch_types=[
            pltpu.VMEM((L,), jnp.int32),
            pltpu.VMEM((L,), jnp.int32),
            pltpu.VMEM((L, W), jnp.float32),
            pltpu.SemaphoreType.DMA,
        ],
    )(_mid_body)
    return k(table, lens).reshape(B, D)


# speedup vs baseline: 7.0425x; 7.0425x over previous
import functools

import jax
import jax.numpy as jnp
from jax import lax
from jax.experimental import pallas as pl
from jax.experimental.pallas import tpu as pltpu
from jax.experimental.pallas import tpu_sc as plsc

B, S, D = 16, 2048, 1024
L = 16


def _mid_body(table_hbm, lens_hbm, out_hbm, lens_v, zrow_v, sem):
    cid = lax.axis_index("c")
    sid = lax.axis_index("s")
    wid = sid + cid

    @pl.when(wid == 0)
    def _():
        copy_lens = pltpu.make_async_copy(lens_hbm, lens_v, sem)
        copy_lens.start()
        zeros = jnp.zeros((L,), jnp.float32)
        for j in range(D // L):
            zrow_v[j * L:(j + 1) * L] = zeros
        copy_lens.wait()
        lens = lens_v[...]
        for b in range(B):
            len_b = lens[b]
            row = (b * S) + lax.shift_right_logical(len_b, 1)

            @pl.when(len_b != 0)
            def _full(row=row, b=b):
                pltpu.make_async_copy(
                    table_hbm.at[row], out_hbm.at[b], sem
                ).start()

            @pl.when(len_b == 0)
            def _empty(b=b):
                pltpu.make_async_copy(zrow_v, out_hbm.at[b], sem).start()

        pltpu.make_async_copy(table_hbm.at[pl.ds(0, B)], out_hbm, sem).wait()


def kernel(payload, seq_lens):
    table = payload.reshape(B * S, D)
    lens = seq_lens.astype(jnp.int32)
    mesh = plsc.VectorSubcoreMesh(
        core_axis_name="c", subcore_axis_name="s", num_cores=1
    )
    k = functools.partial(
        pl.kernel,
        mesh=mesh,
        out_type=jax.ShapeDtypeStruct((B, D), jnp.float32),
        scratch_types=[
            pltpu.VMEM((L,), jnp.int32),
            pltpu.VMEM((D,), jnp.float32),
            pltpu.SemaphoreType.DMA,
        ],
    )(_mid_body)
    return k(table, lens)


# speedup vs baseline: 7.3007x; 1.0367x over previous
import functools

import jax
import jax.numpy as jnp
from jax import lax
from jax.experimental import pallas as pl
from jax.experimental.pallas import tpu as pltpu
from jax.experimental.pallas import tpu_sc as plsc

B, S, D = 16, 2048, 1024
L = 16


def _mid_body(table_hbm, lens_hbm, out_hbm, lens_v, idx_v, rows_v, sem):
    pltpu.sync_copy(lens_hbm, lens_v)
    lens = lens_v[...]
    lane = lax.iota(jnp.int32, L)
    idx_v[...] = lax.shift_right_logical(lens, 1) + lane * S
    pltpu.async_copy(table_hbm.at[idx_v], rows_v, sem).wait()
    zeros = jnp.zeros((L,), jnp.float32)
    for b in range(B):
        @pl.when(lens[b] == 0)
        def _zero(b=b):
            def chunk(j, c):
                rows_v[b, pl.ds(j * L, L)] = zeros
                return c

            lax.fori_loop(0, D // L, chunk, 0)

    pltpu.sync_copy(rows_v, out_hbm)


def kernel(payload, seq_lens):
    table = payload.reshape(B * S, D)
    lens = seq_lens.astype(jnp.int32)
    mesh = plsc.VectorSubcoreMesh(
        core_axis_name="c", subcore_axis_name="s", num_cores=1, num_subcores=1
    )
    k = functools.partial(
        pl.kernel,
        mesh=mesh,
        out_type=jax.ShapeDtypeStruct((B, D), jnp.float32),
        scratch_types=[
            pltpu.VMEM((L,), jnp.int32),
            pltpu.VMEM((L,), jnp.int32),
            pltpu.VMEM((B, D), jnp.float32),
            pltpu.SemaphoreType.DMA,
        ],
    )(_mid_body)
    return k(table, lens)


# speedup vs baseline: 7.4112x; 1.0151x over previous
import functools

import jax
import jax.numpy as jnp
from jax import lax
from jax.experimental import pallas as pl
from jax.experimental.pallas import tpu as pltpu
from jax.experimental.pallas import tpu_sc as plsc

B, S, D = 16, 2048, 1024
L = 16
H = B // 2


def _mid_body(table_hbm, lens_hbm, out_hbm, lens_v, idx_v, rows_v, sem0, sem1):
    pltpu.sync_copy(lens_hbm, lens_v)
    lens = lens_v[...]
    lane = lax.iota(jnp.int32, L)
    idx_v[...] = lax.shift_right_logical(lens, 1) + lane * S
    g0 = pltpu.make_async_copy(
        table_hbm.at[idx_v.at[pl.ds(0, H)]], rows_v.at[pl.ds(0, H)], sem0
    )
    g1 = pltpu.make_async_copy(
        table_hbm.at[idx_v.at[pl.ds(H, H)]], rows_v.at[pl.ds(H, H)], sem1
    )
    g0.start()
    g1.start()
    zeros = jnp.zeros((L,), jnp.float32)

    def zero_rows(lo):
        for b in range(lo, lo + H):
            @pl.when(lens[b] == 0)
            def _zero(b=b):
                def chunk(j, c):
                    rows_v[b, pl.ds(j * L, L)] = zeros
                    return c

                lax.fori_loop(0, D // L, chunk, 0)

    g0.wait()
    zero_rows(0)
    w0 = pltpu.make_async_copy(
        rows_v.at[pl.ds(0, H)], out_hbm.at[pl.ds(0, H)], sem0
    )
    w0.start()
    g1.wait()
    zero_rows(H)
    w1 = pltpu.make_async_copy(
        rows_v.at[pl.ds(H, H)], out_hbm.at[pl.ds(H, H)], sem1
    )
    w1.start()
    w0.wait()
    w1.wait()


def kernel(payload, seq_lens):
    table = payload.reshape(B * S, D)
    lens = seq_lens.astype(jnp.int32)
    mesh = plsc.VectorSubcoreMesh(
        core_axis_name="c", subcore_axis_name="s", num_cores=1, num_subcores=1
    )
    k = functools.partial(
        pl.kernel,
        mesh=mesh,
        out_type=jax.ShapeDtypeStruct((B, D), jnp.float32),
        scratch_types=[
            pltpu.VMEM((L,), jnp.int32),
            pltpu.VMEM((L,), jnp.int32),
            pltpu.VMEM((B, D), jnp.float32),
            pltpu.SemaphoreType.DMA,
            pltpu.SemaphoreType.DMA,
        ],
    )(_mid_body)
    return k(table, lens)


# speedup vs baseline: 7.4969x; 1.0116x over previous
import functools

import jax
import jax.numpy as jnp
from jax import lax
from jax.experimental import pallas as pl
from jax.experimental.pallas import tpu as pltpu
from jax.experimental.pallas import tpu_sc as plsc

B, S, D = 16, 2048, 1024
L = 16
H = B // 2


def _mid_body(table_hbm, lens_hbm, out_hbm, lens_v, idx_v, rows_v, sem):
    t = lax.axis_index("s")
    base = t * H

    pltpu.sync_copy(lens_hbm, lens_v)
    lens = lens_v[...]
    lane = lax.iota(jnp.int32, L)
    idx_v[...] = lax.shift_right_logical(lens, 1) + lane * S
    pltpu.async_copy(
        table_hbm.at[idx_v.at[pl.ds(base, H)]], rows_v, sem
    ).wait()
    zeros = jnp.zeros((L,), jnp.float32)
    for i in range(H):
        my_len = jnp.where(t == 0, lens[i], lens[i + H])

        @pl.when(my_len == 0)
        def _zero(i=i):
            def chunk(j, c):
                rows_v[i, pl.ds(j * L, L)] = zeros
                return c

            lax.fori_loop(0, D // L, chunk, 0)

    pltpu.sync_copy(rows_v, out_hbm.at[pl.ds(base, H)])


def kernel(payload, seq_lens):
    table = payload.reshape(B * S, D)
    lens = seq_lens.astype(jnp.int32)
    mesh = plsc.VectorSubcoreMesh(
        core_axis_name="c", subcore_axis_name="s", num_cores=1, num_subcores=2
    )
    k = functools.partial(
        pl.kernel,
        mesh=mesh,
        out_type=jax.ShapeDtypeStruct((B, D), jnp.float32),
        scratch_types=[
            pltpu.VMEM((L,), jnp.int32),
            pltpu.VMEM((L,), jnp.int32),
            pltpu.VMEM((H, D), jnp.float32),
            pltpu.SemaphoreType.DMA,
        ],
    )(_mid_body)
    return k(table, lens)


# speedup vs baseline: 7.6530x; 1.0208x over previous
import functools

import jax
import jax.numpy as jnp
from jax import lax
from jax.experimental import pallas as pl
from jax.experimental.pallas import tpu as pltpu
from jax.experimental.pallas import tpu_sc as plsc

B, S, D = 16, 2048, 1024
L = 16
H = B // 2
C = H // 2

_PERM = [(k // 8) * 4 + (k % 4) for k in range(2 * L)]


def _mid_body(table_hbm, lr_hbm, out_hbm, lr_v, idx_v, rowsa_v, rowsb_v,
              zrow_v, g0, g1, w0, w1):
    t = lax.axis_index("s")
    base = t * H

    pltpu.sync_copy(lr_hbm, lr_v)
    h0 = lr_v[pl.ds(0, L)]
    h1 = lr_v[pl.ds(L, L)]
    r0 = lr_v[pl.ds(2 * L, L)]
    r1 = lr_v[pl.ds(3 * L, L)]
    idx_v[pl.ds(0, L)] = lax.shift_right_logical(h0, 1) + r0
    idx_v[pl.ds(L, L)] = lax.shift_right_logical(h1, 1) + r1
    ibase = t * L
    cp0 = pltpu.async_copy(
        table_hbm.at[idx_v.at[pl.ds(ibase, C)]], rowsa_v, g0
    )
    cp1 = pltpu.async_copy(
        table_hbm.at[idx_v.at[pl.ds(ibase + 8, C)]], rowsb_v, g1
    )
    zeros = jnp.zeros((L,), jnp.float32)

    def zchunk(j, c):
        zrow_v[0, pl.ds(j * L, L)] = zeros
        return c

    lax.fori_loop(0, D // L, zchunk, 0)
    cp0.wait()
    wr0 = pltpu.async_copy(rowsa_v, out_hbm.at[pl.ds(base, C)], w0)
    cp1.wait()
    wr1 = pltpu.async_copy(rowsb_v, out_hbm.at[pl.ds(base + C, C)], w1)
    wr0.wait()
    wr1.wait()
    for i in range(H):
        p = (i // C) * 8 + (i % C)
        my_len = jnp.where(t == 0, h0[p], h1[p])

        @pl.when(my_len == 0)
        def _zero(i=i):
            pltpu.sync_copy(zrow_v, out_hbm.at[pl.ds(base + i, 1)])


def kernel(payload, seq_lens):
    table = payload.reshape(B * S, D)
    lens = seq_lens.astype(jnp.int32)
    perm = jnp.asarray(_PERM, dtype=jnp.int32)
    lens_p = lens[perm]
    rows_p = perm * jnp.int32(S)
    lr = jnp.concatenate([lens_p, rows_p])
    mesh = plsc.VectorSubcoreMesh(
        core_axis_name="c", subcore_axis_name="s", num_cores=1, num_subcores=2
    )
    k = functools.partial(
        pl.kernel,
        mesh=mesh,
        out_type=jax.ShapeDtypeStruct((B, D), jnp.float32),
        scratch_types=[
            pltpu.VMEM((4 * L,), jnp.int32),
            pltpu.VMEM((2 * L,), jnp.int32),
            pltpu.VMEM((C, D), jnp.float32),
            pltpu.VMEM((C, D), jnp.float32),
            pltpu.VMEM((1, D), jnp.float32),
            pltpu.SemaphoreType.DMA,
            pltpu.SemaphoreType.DMA,
            pltpu.SemaphoreType.DMA,
            pltpu.SemaphoreType.DMA,
        ],
    )(_mid_body)
    return k(table, lr)


# speedup vs baseline: 7.7434x; 1.0118x over previous
import functools

import jax
import jax.numpy as jnp
from jax import lax
from jax.experimental import pallas as pl
from jax.experimental.pallas import tpu as pltpu
from jax.experimental.pallas import tpu_sc as plsc

B, S, D = 16, 2048, 1024
L = 16
H = B // 2
C = H // 2

_PERM = [(k // 8) * 4 + (k % 4) for k in range(2 * L)]


def _mid_body(table_hbm, lr_hbm, out_hbm, lr_v, idx_v, rowsa_v, zrow_v,
              g0, w0):
    t = lax.axis_index("s")
    base = t * C

    pltpu.sync_copy(lr_hbm, lr_v)
    h0 = lr_v[pl.ds(0, L)]
    h1 = lr_v[pl.ds(L, L)]
    r0 = lr_v[pl.ds(2 * L, L)]
    r1 = lr_v[pl.ds(3 * L, L)]
    idx_v[pl.ds(0, L)] = lax.shift_right_logical(h0, 1) + r0
    idx_v[pl.ds(L, L)] = lax.shift_right_logical(h1, 1) + r1
    ibase = t * 8
    cp0 = pltpu.async_copy(
        table_hbm.at[idx_v.at[pl.ds(ibase, C)]], rowsa_v, g0
    )
    zeros = jnp.zeros((L,), jnp.float32)

    def zchunk(j, c):
        zrow_v[0, pl.ds(j * L, L)] = zeros
        return c

    lax.fori_loop(0, D // L, zchunk, 0)
    cp0.wait()
    wr0 = pltpu.async_copy(rowsa_v, out_hbm.at[pl.ds(base, C)], w0)
    wr0.wait()
    for i in range(C):
        my_len = jnp.where(
            t == 0, h0[i],
            jnp.where(t == 1, h0[8 + i], jnp.where(t == 2, h1[i], h1[8 + i])),
        )

        @pl.when(my_len == 0)
        def _zero(i=i):
            pltpu.sync_copy(zrow_v, out_hbm.at[pl.ds(base + i, 1)])


def kernel(payload, seq_lens):
    table = payload.reshape(B * S, D)
    lens = seq_lens.astype(jnp.int32)
    perm = jnp.asarray(_PERM, dtype=jnp.int32)
    lens_p = lens[perm]
    rows_p = perm * jnp.int32(S)
    lr = jnp.concatenate([lens_p, rows_p])
    mesh = plsc.VectorSubcoreMesh(
        core_axis_name="c", subcore_axis_name="s", num_cores=1, num_subcores=4
    )
    k = functools.partial(
        pl.kernel,
        mesh=mesh,
        out_type=jax.ShapeDtypeStruct((B, D), jnp.float32),
        scratch_types=[
            pltpu.VMEM((4 * L,), jnp.int32),
            pltpu.VMEM((2 * L,), jnp.int32),
            pltpu.VMEM((C, D), jnp.float32),
            pltpu.VMEM((1, D), jnp.float32),
            pltpu.SemaphoreType.DMA,
            pltpu.SemaphoreType.DMA,
        ],
    )(_mid_body)
    return k(table, lr)


# speedup vs baseline: 7.7701x; 1.0034x over previous
import functools

import jax
import jax.numpy as jnp
from jax import lax
from jax.experimental import pallas as pl
from jax.experimental.pallas import tpu as pltpu
from jax.experimental.pallas import tpu_sc as plsc

B, S, D = 16, 2048, 1024
L = 16
H = B // 2
C = 2

_PERM = [(k // 8) * 2 + (k % 2) for k in range(4 * L)]


def _mid_body(table_hbm, lr_hbm, out_hbm, lr_v, idx_v, rowsa_v, zrow_v,
              g0, w0):
    t = lax.axis_index("s")
    base = t * C

    pltpu.sync_copy(lr_hbm, lr_v)
    hs = [lr_v[pl.ds(s * L, L)] for s in range(4)]
    rs = [lr_v[pl.ds((4 + s) * L, L)] for s in range(4)]
    for s in range(4):
        idx_v[pl.ds(s * L, L)] = lax.shift_right_logical(hs[s], 1) + rs[s]
    ibase = t * 8
    cp0 = pltpu.async_copy(
        table_hbm.at[idx_v.at[pl.ds(ibase, C)]], rowsa_v, g0
    )
    zeros = jnp.zeros((L,), jnp.float32)

    def zchunk(j, c):
        zrow_v[0, pl.ds(j * L, L)] = zeros
        return c

    lax.fori_loop(0, D // L, zchunk, 0)
    cp0.wait()
    wr0 = pltpu.async_copy(rowsa_v, out_hbm.at[pl.ds(base, C)], w0)
    wr0.wait()
    for i in range(C):
        my_len = hs[3][8 + i]
        for tt in range(6, -1, -1):
            my_len = jnp.where(t == tt, hs[tt // 2][(tt % 2) * 8 + i], my_len)

        @pl.when(my_len == 0)
        def _zero(i=i):
            pltpu.sync_copy(zrow_v, out_hbm.at[pl.ds(base + i, 1)])


def kernel(payload, seq_lens):
    table = payload.reshape(B * S, D)
    lens = seq_lens.astype(jnp.int32)
    perm = jnp.asarray(_PERM, dtype=jnp.int32)
    lens_p = lens[perm]
    rows_p = perm * jnp.int32(S)
    lr = jnp.concatenate([lens_p, rows_p])
    mesh = plsc.VectorSubcoreMesh(
        core_axis_name="c", subcore_axis_name="s", num_cores=1, num_subcores=8
    )
    k = functools.partial(
        pl.kernel,
        mesh=mesh,
        out_type=jax.ShapeDtypeStruct((B, D), jnp.float32),
        scratch_types=[
            pltpu.VMEM((8 * L,), jnp.int32),
            pltpu.VMEM((4 * L,), jnp.int32),
            pltpu.VMEM((C, D), jnp.float32),
            pltpu.VMEM((1, D), jnp.float32),
            pltpu.SemaphoreType.DMA,
            pltpu.SemaphoreType.DMA,
        ],
    )(_mid_body)
    return k(table, lr)
